# baseline (speedup 1.0000x reference)
import jax
import jax.numpy as jnp
from jax.experimental import pallas as pl
from jax.experimental.pallas import tpu as pltpu

B = 256
A = 64
DIM_IN = 256
N_CART = 3 * A
REST = DIM_IN - N_CART
H = 64
GOUT = 64
RMAX = 1.5

NB = 64
NB2 = NB // 2


def _pair(q0, q1):
    return jnp.concatenate(
        [jnp.broadcast_to(q0[:, :, None], (NB2, A, A)),
         jnp.broadcast_to(q1[:, :, None], (NB2, A, A))], axis=-1)


def _gnn_kernel(px_ref, py_ref, pz_ref,
                te_ref, wmsg_ref, wgeo_sm, bmsg_ref, wnode_ref, bnode_ref,
                out_ref,
                dx_ref, dy_ref, dz_ref, dd_ref, mk_ref, tb_ref, acc_ref):
    px = px_ref[...].reshape(NB2, 2, A)
    py = py_ref[...].reshape(NB2, 2, A)
    pz = pz_ref[...].reshape(NB2, 2, A)
    pxe, pxo = px[:, 0, :], px[:, 1, :]
    pye, pyo = py[:, 0, :], py[:, 1, :]
    pze, pzo = pz[:, 0, :], pz[:, 1, :]
    pxd = jnp.concatenate([pxe, pxo], axis=-1)[:, None, :]
    pyd = jnp.concatenate([pye, pyo], axis=-1)[:, None, :]
    pzd = jnp.concatenate([pze, pzo], axis=-1)[:, None, :]
    dxv = pxd - _pair(pxe, pxo)
    dyv = pyd - _pair(pye, pyo)
    dzv = pzd - _pair(pze, pzo)
    dv = jnp.sqrt(dxv * dxv + dyv * dyv + dzv * dzv)
    ii = jax.lax.broadcasted_iota(jnp.int32, (A, A), 0)
    jj = jax.lax.broadcasted_iota(jnp.int32, (A, A), 1)
    offd = (ii != jj).astype(jnp.float32)
    offd2 = jnp.concatenate([offd, offd], axis=-1)[None]
    mkv = jnp.where(dv <= RMAX, 1.0, 0.0) * offd2
    mk_ref[...] = mkv
    dx_ref[...] = dxv * mkv
    dy_ref[...] = dyv * mkv
    dz_ref[...] = dzv * mkv
    dd_ref[...] = dv * mkv

    te = te_ref[...]
    t_all = (jnp.dot(te, wmsg_ref[:H, :],
                     preferred_element_type=jnp.float32)
             + bmsg_ref[...])
    tb_ref[...] = jnp.broadcast_to(t_all.T[:, :, None], (H, A, 2 * A))

    def body(h, _):
        w0 = wgeo_sm[0, h]
        w1 = wgeo_sm[1, h]
        w2 = wgeo_sm[2, h]
        w3 = wgeo_sm[3, h]
        pre = (tb_ref[h] * mk_ref[...]
               + w0 * dx_ref[...]
               + w1 * dy_ref[...]
               + w2 * dz_ref[...]
               + w3 * dd_ref[...])
        msg = jnp.maximum(pre, 0.0)
        acc_ref[pl.ds(h, 1)] = jnp.sum(msg, axis=1)[None]
        return 0

    jax.lax.fori_loop(0, H, body, 0, unroll=4)

    te2 = jnp.dot(te, wnode_ref[:H, :], preferred_element_type=jnp.float32)
    te2p = jnp.concatenate([te2, te2], axis=0)
    wn2 = wnode_ref[H:, :]
    bn = bnode_ref[...]
    for b2 in range(NB2):
        cols = acc_ref[:, b2, :]
        g = jax.lax.dot_general(cols, wn2, (((0,), (0,)), ((), ())),
                                preferred_element_type=jnp.float32)
        g = jnp.maximum(g + te2p + bn, 0.0)
        out_ref[pl.ds(2 * b2, 2)] = g.reshape(2, A, GOUT)

    return


def _mlp_kernel(feat_ref, w1_ref, b1_ref, w2_ref, b2_ref, w3_ref, b3_ref,
                out_ref):
    h = jnp.dot(feat_ref[...], w1_ref[...], preferred_element_type=jnp.float32)
    h = jnp.maximum(h + b1_ref[...], 0.0)
    h = jnp.dot(h, w2_ref[...], preferred_element_type=jnp.float32)
    h = jnp.maximum(h + b2_ref[...], 0.0)
    out_ref[...] = (jnp.dot(h, w3_ref[...], preferred_element_type=jnp.float32)
                    + b3_ref[...])


def kernel(x, type_embed, W_msg, b_msg, W_node, b_node, W1, b1, W2, b2, W3, b3):
    x_rest = x[:, :REST]
    x_cart = x[:, REST:].reshape(B, A, 3)
    px = x_cart[:, :, 0]
    py = x_cart[:, :, 1]
    pz = x_cart[:, :, 2]
    wgeo = W_msg[H:H + 4, :]

    grid = (B // NB,)
    gnn = pl.pallas_call(
        _gnn_kernel,
        grid=grid,
        in_specs=[
            pl.BlockSpec((NB, A), lambda i: (i, 0)),
            pl.BlockSpec((NB, A), lambda i: (i, 0)),
            pl.BlockSpec((NB, A), lambda i: (i, 0)),
            pl.BlockSpec((A, H), lambda i: (0, 0)),
            pl.BlockSpec((H + 4, H), lambda i: (0, 0)),
            pl.BlockSpec(memory_space=pltpu.SMEM),
            pl.BlockSpec((1, H), lambda i: (0, 0)),
            pl.BlockSpec((2 * H, GOUT), lambda i: (0, 0)),
            pl.BlockSpec((1, GOUT), lambda i: (0, 0)),
        ],
        out_specs=pl.BlockSpec((NB, A, GOUT), lambda i: (i, 0, 0)),
        out_shape=jax.ShapeDtypeStruct((B, A, GOUT), jnp.float32),
        compiler_params=pltpu.CompilerParams(
            dimension_semantics=("parallel",)),
        scratch_shapes=[pltpu.VMEM((NB2, A, 2 * A), jnp.float32)] * 5
        + [pltpu.VMEM((H, A, 2 * A), jnp.float32),
           pltpu.VMEM((H, NB2, 2 * A), jnp.float32)],
    )(px, py, pz, type_embed, W_msg, wgeo, b_msg.reshape(1, H),
      W_node, b_node.reshape(1, GOUT))

    feat = jnp.concatenate([x_rest, gnn.reshape(B, A * GOUT)], axis=1)

    out = pl.pallas_call(
        _mlp_kernel,
        out_shape=jax.ShapeDtypeStruct((B, W3.shape[1]), jnp.float32),
    )(feat, W1, b1.reshape(1, -1), W2, b2.reshape(1, -1), W3,
      b3.reshape(1, -1))
    return out


# speedup vs baseline: 1.0362x; 1.0362x over previous
import jax
import jax.numpy as jnp
from jax.experimental import pallas as pl
from jax.experimental.pallas import tpu as pltpu

B = 256
A = 64
DIM_IN = 256
N_CART = 3 * A
REST = DIM_IN - N_CART
H = 64
GOUT = 64
RMAX = 1.5

NB = 128
NB2 = NB // 2


def _pair(q0, q1):
    return jnp.concatenate(
        [jnp.broadcast_to(q0[:, :, None], (NB2, A, A)),
         jnp.broadcast_to(q1[:, :, None], (NB2, A, A))], axis=-1)


def _gnn_kernel(px_ref, py_ref, pz_ref,
                te_ref, wmsg_ref, wgeo_sm, bmsg_ref, wnode_ref, bnode_ref,
                out_ref,
                dx_ref, dy_ref, dz_ref, dd_ref, mk_ref, tb_ref, acc_ref):
    px = px_ref[...].reshape(NB2, 2, A)
    py = py_ref[...].reshape(NB2, 2, A)
    pz = pz_ref[...].reshape(NB2, 2, A)
    pxe, pxo = px[:, 0, :], px[:, 1, :]
    pye, pyo = py[:, 0, :], py[:, 1, :]
    pze, pzo = pz[:, 0, :], pz[:, 1, :]
    pxd = jnp.concatenate([pxe, pxo], axis=-1)[:, None, :]
    pyd = jnp.concatenate([pye, pyo], axis=-1)[:, None, :]
    pzd = jnp.concatenate([pze, pzo], axis=-1)[:, None, :]
    dxv = pxd - _pair(pxe, pxo)
    dyv = pyd - _pair(pye, pyo)
    dzv = pzd - _pair(pze, pzo)
    dv = jnp.sqrt(dxv * dxv + dyv * dyv + dzv * dzv)
    ii = jax.lax.broadcasted_iota(jnp.int32, (A, A), 0)
    jj = jax.lax.broadcasted_iota(jnp.int32, (A, A), 1)
    offd = (ii != jj).astype(jnp.float32)
    offd2 = jnp.concatenate([offd, offd], axis=-1)[None]
    mk_ref[...] = jnp.where(dv <= RMAX, 1.0, 0.0) * offd2
    dx_ref[...] = dxv
    dy_ref[...] = dyv
    dz_ref[...] = dzv
    dd_ref[...] = dv

    te = te_ref[...]
    t_all = (jnp.dot(te, wmsg_ref[:H, :],
                     preferred_element_type=jnp.float32)
             + bmsg_ref[...])
    tb_ref[...] = jnp.broadcast_to(t_all.T[:, :, None], (H, A, 2 * A))

    def body(h, _):
        w0 = wgeo_sm[0, h]
        w1 = wgeo_sm[1, h]
        w2 = wgeo_sm[2, h]
        w3 = wgeo_sm[3, h]
        pre = (tb_ref[h]
               + w0 * dx_ref[...]
               + w1 * dy_ref[...]
               + w2 * dz_ref[...]
               + w3 * dd_ref[...])
        msg = jnp.maximum(pre, 0.0) * mk_ref[...]
        acc_ref[pl.ds(h, 1)] = jnp.sum(msg, axis=1)[None]
        return 0

    jax.lax.fori_loop(0, H, body, 0, unroll=4)

    te2 = jnp.dot(te, wnode_ref[:H, :], preferred_element_type=jnp.float32)
    te2p = jnp.concatenate([te2, te2], axis=0)
    wn2 = wnode_ref[H:, :]
    bn = bnode_ref[...]
    for b2 in range(NB2):
        cols = acc_ref[:, b2, :]
        g = jax.lax.dot_general(cols, wn2, (((0,), (0,)), ((), ())),
                                preferred_element_type=jnp.float32)
        g = jnp.maximum(g + te2p + bn, 0.0)
        out_ref[pl.ds(2 * b2, 2)] = g.reshape(2, A, GOUT)

    return


def _mlp_kernel(feat_ref, w1_ref, b1_ref, w2_ref, b2_ref, w3_ref, b3_ref,
                out_ref):
    h = jnp.dot(feat_ref[...], w1_ref[...], preferred_element_type=jnp.float32)
    h = jnp.maximum(h + b1_ref[...], 0.0)
    h = jnp.dot(h, w2_ref[...], preferred_element_type=jnp.float32)
    h = jnp.maximum(h + b2_ref[...], 0.0)
    out_ref[...] = (jnp.dot(h, w3_ref[...], preferred_element_type=jnp.float32)
                    + b3_ref[...])


def kernel(x, type_embed, W_msg, b_msg, W_node, b_node, W1, b1, W2, b2, W3, b3):
    x_rest = x[:, :REST]
    x_cart = x[:, REST:].reshape(B, A, 3)
    px = x_cart[:, :, 0]
    py = x_cart[:, :, 1]
    pz = x_cart[:, :, 2]
    wgeo = W_msg[H:H + 4, :]

    grid = (B // NB,)
    gnn = pl.pallas_call(
        _gnn_kernel,
        grid=grid,
        in_specs=[
            pl.BlockSpec((NB, A), lambda i: (i, 0)),
            pl.BlockSpec((NB, A), lambda i: (i, 0)),
            pl.BlockSpec((NB, A), lambda i: (i, 0)),
            pl.BlockSpec((A, H), lambda i: (0, 0)),
            pl.BlockSpec((H + 4, H), lambda i: (0, 0)),
            pl.BlockSpec(memory_space=pltpu.SMEM),
            pl.BlockSpec((1, H), lambda i: (0, 0)),
            pl.BlockSpec((2 * H, GOUT), lambda i: (0, 0)),
            pl.BlockSpec((1, GOUT), lambda i: (0, 0)),
        ],
        out_specs=pl.BlockSpec((NB, A, GOUT), lambda i: (i, 0, 0)),
        out_shape=jax.ShapeDtypeStruct((B, A, GOUT), jnp.float32),
        compiler_params=pltpu.CompilerParams(
            dimension_semantics=("parallel",)),
        scratch_shapes=[pltpu.VMEM((NB2, A, 2 * A), jnp.float32)] * 5
        + [pltpu.VMEM((H, A, 2 * A), jnp.float32),
           pltpu.VMEM((H, NB2, 2 * A), jnp.float32)],
    )(px, py, pz, type_embed, W_msg, wgeo, b_msg.reshape(1, H),
      W_node, b_node.reshape(1, GOUT))

    feat = jnp.concatenate([x_rest, gnn.reshape(B, A * GOUT)], axis=1)

    out = pl.pallas_call(
        _mlp_kernel,
        out_shape=jax.ShapeDtypeStruct((B, W3.shape[1]), jnp.float32),
    )(feat, W1, b1.reshape(1, -1), W2, b2.reshape(1, -1), W3,
      b3.reshape(1, -1))
    return out
